# Initial kernel scaffold; baseline (speedup 1.0000x reference)
import jax
import jax.numpy as jnp
from jax.experimental import pallas as pl


def kernel(input_ids, tok_emb, pos_emb, gamma, beta):
    raise NotImplementedError("write your pallas kernel here")


# speedup vs baseline: 5.2345x; 5.2345x over previous
import functools

import jax
import jax.numpy as jnp
from jax.experimental import pallas as pl

_EPS = 1e-12


def _embed_ln_kernel(ids_ref, tok_ref, pos_ref, gamma_ref, beta_ref, out_ref,
                     *, vocab: int):
    b, sblk, _ = ids_ref.shape
    d = tok_ref.shape[1]
    tok_tab = tok_ref[...]
    pos = pos_ref[...]
    g = gamma_ref[...]
    bt = beta_ref[...]
    iota = jax.lax.broadcasted_iota(jnp.int32, (sblk, vocab), 1)
    for bi in range(b):
        ids = ids_ref[bi]
        onehot = (ids == iota).astype(jnp.float32)
        x = jnp.dot(onehot, tok_tab, preferred_element_type=jnp.float32) + pos
        mean = jnp.mean(x, axis=-1, keepdims=True)
        xc = x - mean
        var = jnp.mean(xc * xc, axis=-1, keepdims=True)
        xhat = xc * jax.lax.rsqrt(var + _EPS)
        out_ref[bi] = xhat * g + bt


def kernel(input_ids, tok_emb, pos_emb, gamma, beta):
    b, s = input_ids.shape
    vocab, d = tok_emb.shape
    sblk = 512
    grid = (s // sblk,)

    ids = input_ids.astype(jnp.int32).reshape(b, s, 1)
    pos = pos_emb[:s]

    out = pl.pallas_call(
        functools.partial(_embed_ln_kernel, vocab=vocab),
        grid=grid,
        in_specs=[
            pl.BlockSpec((b, sblk, 1), lambda i: (0, i, 0)),
            pl.BlockSpec((vocab, d), lambda i: (0, 0)),
            pl.BlockSpec((sblk, d), lambda i: (i, 0)),
            pl.BlockSpec((d,), lambda i: (0,)),
            pl.BlockSpec((d,), lambda i: (0,)),
        ],
        out_specs=pl.BlockSpec((b, sblk, d), lambda i: (0, i, 0)),
        out_shape=jax.ShapeDtypeStruct((b, s, d), jnp.float32),
    )(ids, tok_emb, pos, gamma, beta)
    return out


# speedup vs baseline: 5.4136x; 1.0342x over previous
import functools

import jax
import jax.numpy as jnp
from jax.experimental import pallas as pl

_EPS = 1e-12


def _embed_ln_kernel(ids_ref, tok_ref, pos_ref, gamma_ref, beta_ref, out_ref,
                     *, vocab: int):
    b, sblk, _ = ids_ref.shape
    d = tok_ref.shape[1]
    tok_tab = tok_ref[...]
    pos = pos_ref[...]
    g = gamma_ref[...]
    bt = beta_ref[...]
    iota = jax.lax.broadcasted_iota(jnp.int32, (sblk, vocab), 1)
    for bi in range(b):
        ids = ids_ref[bi]
        onehot = (ids == iota).astype(jnp.float32)
        x = jnp.dot(onehot, tok_tab, preferred_element_type=jnp.float32) + pos
        mean = jnp.mean(x, axis=-1, keepdims=True)
        xc = x - mean
        var = jnp.mean(xc * xc, axis=-1, keepdims=True)
        xhat = xc * jax.lax.rsqrt(var + _EPS)
        out_ref[bi] = xhat * g + bt


def kernel(input_ids, tok_emb, pos_emb, gamma, beta):
    b, s = input_ids.shape
    vocab, d = tok_emb.shape
    sblk = 1024
    grid = (s // sblk,)

    ids = input_ids.astype(jnp.int32).reshape(b, s, 1)
    pos = pos_emb[:s]

    out = pl.pallas_call(
        functools.partial(_embed_ln_kernel, vocab=vocab),
        grid=grid,
        in_specs=[
            pl.BlockSpec((b, sblk, 1), lambda i: (0, i, 0)),
            pl.BlockSpec((vocab, d), lambda i: (0, 0)),
            pl.BlockSpec((sblk, d), lambda i: (i, 0)),
            pl.BlockSpec((d,), lambda i: (0,)),
            pl.BlockSpec((d,), lambda i: (0,)),
        ],
        out_specs=pl.BlockSpec((b, sblk, d), lambda i: (0, i, 0)),
        out_shape=jax.ShapeDtypeStruct((b, s, d), jnp.float32),
    )(ids, tok_emb, pos, gamma, beta)
    return out
